# baseline (speedup 1.0000x reference)
import functools

import jax
import jax.numpy as jnp
from jax import lax
from jax.experimental import pallas as pl
from jax.experimental.pallas import tpu as pltpu
from jax.experimental.pallas import tpu_sc as plsc

_LANES = 16
_PARAMS = pltpu.CompilerParams(
    needs_layout_passes=False, use_tc_tiling_on_sc=False)


def _flat_view(w):
    v, d = w.shape
    nb = v // 128
    t = w.T[:, : nb * 128]
    t = t.reshape(d // 8, 8, nb, 128).transpose(0, 2, 1, 3)
    return t.reshape(-1)


def _mesh():
    return plsc.VectorSubcoreMesh(core_axis_name="c", subcore_axis_name="s")


@functools.lru_cache(maxsize=None)
def _build_k1(batch, dim, ivocab):
    info = plsc.get_sparse_core_info()
    nc, ns = info.num_cores, info.num_subcores
    nw = nc * ns
    bpw = batch // nw
    groups = bpw // _LANES
    inb = ivocab // 128
    itrunc = inb * 128
    itail = ivocab - itrunc

    @functools.partial(
        pl.kernel,
        mesh=_mesh(),
        compiler_params=_PARAMS,
        out_type=(jax.ShapeDtypeStruct((dim, batch), jnp.float32),
                  jax.ShapeDtypeStruct((batch,), jnp.float32)),
        scratch_types=[
            pltpu.VMEM((bpw,), jnp.int32),
            pltpu.VMEM((bpw,), jnp.int32),
            pltpu.VMEM((bpw,), jnp.int32),
            pltpu.VMEM((dim, bpw), jnp.float32),
            pltpu.VMEM((max(itail, 8), dim), jnp.float32),
            pltpu.VMEM((bpw,), jnp.float32),
            pltpu.VMEM((bpw,), jnp.float32),
            pltpu.SemaphoreType.DMA,
        ],
    )
    def k1(user_hbm, item_hbm, iflat_hbm, iside_hbm, ubias_hbm, ibias_hbm,
           ie_out_hbm, bias_out_hbm,
           uidx_v, iidx_v, ioff_v, ie_v, iside_v, ub_v, ib_v, sem):
        wid = lax.axis_index("s") * nc + lax.axis_index("c")
        base = wid * bpw
        pltpu.sync_copy(user_hbm.at[pl.ds(base, bpw)], uidx_v)
        pltpu.sync_copy(item_hbm.at[pl.ds(base, bpw)], iidx_v)

        def offsets(g, _):
            sl = pl.ds(g * _LANES, _LANES)
            ri = jnp.minimum(iidx_v[sl], itrunc - 1)
            ioff_v[sl] = (ri >> 7) * 1024 + (ri & 127)
            return _

        lax.fori_loop(0, groups, offsets, 0)

        copies = [
            pltpu.async_copy(iside_hbm, iside_v.at[pl.ds(0, itail)], sem),
            pltpu.async_copy(ubias_hbm.at[0].at[uidx_v], ub_v, sem),
            pltpu.async_copy(ibias_hbm.at[0].at[iidx_v], ib_v, sem),
        ]
        for d in range(dim):
            ci = (d // 8) * inb * 1024 + (d % 8) * 128
            span_i = inb * 1024 - (d % 8) * 128
            copies.append(pltpu.async_copy(
                iflat_hbm.at[pl.ds(ci, span_i)].at[ioff_v], ie_v.at[d], sem))
        for cp in copies:
            cp.wait()

        def group(g, carry):
            sl = pl.ds(g * _LANES, _LANES)
            ri = iidx_v[sl]
            imask = ri >= itrunc
            si = jnp.where(imask, ri - itrunc, 0)
            for d in range(dim):
                dcol = jnp.full((_LANES,), d, jnp.int32)
                ival = jnp.where(imask, plsc.load_gather(iside_v, [si, dcol]),
                                 ie_v[d, sl])
                ie_v[d, sl] = ival
            ub_v[sl] = ub_v[sl] + ib_v[sl]
            return carry

        lax.fori_loop(0, groups, group, 0)
        for d in range(dim):
            pltpu.sync_copy(ie_v.at[d], ie_out_hbm.at[d, pl.ds(base, bpw)])
        pltpu.sync_copy(ub_v, bias_out_hbm.at[pl.ds(base, bpw)])

    return k1


@functools.lru_cache(maxsize=None)
def _build_k2(batch, dim, uvocab):
    info = plsc.get_sparse_core_info()
    nc, ns = info.num_cores, info.num_subcores
    nw = nc * ns
    bpw = batch // nw
    groups = bpw // _LANES
    unb = uvocab // 128
    utrunc = unb * 128
    utail = uvocab - utrunc
    words = dim // 2

    @functools.partial(
        pl.kernel,
        mesh=_mesh(),
        compiler_params=_PARAMS,
        out_type=jax.ShapeDtypeStruct((batch,), jnp.float32),
        scratch_types=[
            pltpu.VMEM((bpw,), jnp.int32),
            pltpu.VMEM((bpw,), jnp.int32),
            pltpu.VMEM((bpw, words), jnp.float32),
            pltpu.VMEM((dim, bpw), jnp.float32),
            pltpu.VMEM((max(utail, 8), dim), jnp.float32),
            pltpu.VMEM((bpw,), jnp.float32),
            pltpu.VMEM((bpw,), jnp.float32),
            pltpu.SemaphoreType.DMA,
        ],
    )
    def k2(user_hbm, urows_hbm, uside_hbm, ie_hbm, bias_hbm, out_hbm,
           uidx_v, uoff_v, ue_v, ie_v, uside_v, bs_v, out_v, sem):
        wid = lax.axis_index("s") * nc + lax.axis_index("c")
        base = wid * bpw
        pltpu.sync_copy(user_hbm.at[pl.ds(base, bpw)], uidx_v)

        def offsets(g, _):
            sl = pl.ds(g * _LANES, _LANES)
            uoff_v[sl] = jnp.minimum(uidx_v[sl], utrunc - 1)
            return _

        lax.fori_loop(0, groups, offsets, 0)

        copies = [
            pltpu.async_copy(uside_hbm, uside_v.at[pl.ds(0, utail)], sem),
            pltpu.async_copy(bias_hbm.at[pl.ds(base, bpw)], bs_v, sem),
            pltpu.async_copy(urows_hbm.at[uoff_v], ue_v, sem),
        ]
        for d in range(dim):
            copies.append(pltpu.async_copy(
                ie_hbm.at[d, pl.ds(base, bpw)], ie_v.at[d], sem))
        for cp in copies:
            cp.wait()

        def group(g, carry):
            sl = pl.ds(g * _LANES, _LANES)
            rows = lax.iota(jnp.int32, _LANES) + g * _LANES
            ru = uidx_v[sl]
            umask = ru >= utrunc
            su = jnp.where(umask, ru - utrunc, 0)
            acc = bs_v[sl]
            for k in range(words):
                kcol = jnp.full((_LANES,), k, jnp.int32)
                pw = plsc.load_gather(ue_v, [rows, kcol])
                ua, ub = plsc.unpack(plsc.bitcast(pw, jnp.bfloat16),
                                     format=plsc.PackFormat.INTERLEAVED)
                ca = jnp.full((_LANES,), 2 * k, jnp.int32)
                cb = jnp.full((_LANES,), 2 * k + 1, jnp.int32)
                uva = jnp.where(umask, plsc.load_gather(uside_v, [su, ca]), ua)
                uvb = jnp.where(umask, plsc.load_gather(uside_v, [su, cb]), ub)
                acc = acc + uva * ie_v[2 * k, sl] + uvb * ie_v[2 * k + 1, sl]
            out_v[sl] = acc
            return carry

        lax.fori_loop(0, groups, group, 0)
        pltpu.sync_copy(out_v, out_hbm.at[pl.ds(base, bpw)])

    return k2


def kernel(user, item, user_emb_w, item_emb_w, user_bias_w, item_bias_w):
    batch = user.shape[0]
    uvocab, dim = user_emb_w.shape
    ivocab = item_emb_w.shape[0]
    user = user.astype(jnp.int32)
    item = item.astype(jnp.int32)
    ie_staged, bias_sum = _build_k1(batch, dim, ivocab)(
        user, item, _flat_view(item_emb_w),
        item_emb_w[(ivocab // 128) * 128:, :],
        user_bias_w.T, item_bias_w.T)
    utrunc = (uvocab // 128) * 128
    urows = lax.bitcast_convert_type(
        user_emb_w[:utrunc].astype(jnp.bfloat16).reshape(utrunc, dim // 2, 2),
        jnp.float32)
    return _build_k2(batch, dim, uvocab)(
        user, urows.reshape(utrunc, dim // 2),
        user_emb_w[utrunc:, :],
        ie_staged, bias_sum)


# speedup vs baseline: 5.5413x; 5.5413x over previous
import functools

import jax
import jax.numpy as jnp
from jax import lax
from jax.experimental import pallas as pl
from jax.experimental.pallas import tpu as pltpu
from jax.experimental.pallas import tpu_sc as plsc

_LANES = 16
_PARAMS = pltpu.CompilerParams(
    needs_layout_passes=False, use_tc_tiling_on_sc=False)


def _flat_view(w):
    v, d = w.shape
    nb = v // 128
    t = w.T[:, : nb * 128]
    t = t.reshape(d // 8, 8, nb, 128).transpose(0, 2, 1, 3)
    return t.reshape(-1)


def _mesh():
    return plsc.VectorSubcoreMesh(core_axis_name="c", subcore_axis_name="s")


@functools.lru_cache(maxsize=None)
def _build_k1(batch, dim, ivocab):
    info = plsc.get_sparse_core_info()
    nc, ns = info.num_cores, info.num_subcores
    nw = nc * ns
    bpw = batch // nw
    groups = bpw // _LANES
    inb = ivocab // 128
    itrunc = inb * 128
    itail = ivocab - itrunc

    @functools.partial(
        pl.kernel,
        mesh=_mesh(),
        compiler_params=_PARAMS,
        out_type=(jax.ShapeDtypeStruct((dim, batch), jnp.float32),
                  jax.ShapeDtypeStruct((batch,), jnp.float32)),
        scratch_types=[
            pltpu.VMEM((bpw,), jnp.int32),
            pltpu.VMEM((bpw,), jnp.int32),
            pltpu.VMEM((bpw,), jnp.int32),
            pltpu.VMEM((dim, bpw), jnp.float32),
            pltpu.VMEM((max(itail, 8), dim), jnp.float32),
            pltpu.VMEM((bpw,), jnp.float32),
            pltpu.VMEM((bpw,), jnp.float32),
            pltpu.SemaphoreType.DMA,
        ],
    )
    def k1(user_hbm, item_hbm, iflat_hbm, iside_hbm, ubias_hbm, ibias_hbm,
           ie_out_hbm, bias_out_hbm,
           uidx_v, iidx_v, ioff_v, ie_v, iside_v, ub_v, ib_v, sem):
        wid = lax.axis_index("s") * nc + lax.axis_index("c")
        base = wid * bpw
        pltpu.sync_copy(user_hbm.at[pl.ds(base, bpw)], uidx_v)
        pltpu.sync_copy(item_hbm.at[pl.ds(base, bpw)], iidx_v)

        def offsets(g, _):
            sl = pl.ds(g * _LANES, _LANES)
            ri = jnp.minimum(iidx_v[sl], itrunc - 1)
            ioff_v[sl] = (ri >> 7) * 1024 + (ri & 127)
            return _

        lax.fori_loop(0, groups, offsets, 0)

        copies = [
            pltpu.async_copy(iside_hbm, iside_v.at[pl.ds(0, itail)], sem),
            pltpu.async_copy(ubias_hbm.at[0].at[uidx_v], ub_v, sem),
            pltpu.async_copy(ibias_hbm.at[0].at[iidx_v], ib_v, sem),
        ]
        for d in range(dim):
            ci = (d // 8) * inb * 1024 + (d % 8) * 128
            span_i = inb * 1024 - (d % 8) * 128
            copies.append(pltpu.async_copy(
                iflat_hbm.at[pl.ds(ci, span_i)].at[ioff_v], ie_v.at[d], sem))
        for cp in copies:
            cp.wait()

        def group(g, carry):
            sl = pl.ds(g * _LANES, _LANES)
            ri = iidx_v[sl]
            imask = ri >= itrunc
            si = jnp.where(imask, ri - itrunc, 0)
            for d in range(dim):
                dcol = jnp.full((_LANES,), d, jnp.int32)
                ival = jnp.where(imask, plsc.load_gather(iside_v, [si, dcol]),
                                 ie_v[d, sl])
                ie_v[d, sl] = ival
            ub_v[sl] = ub_v[sl] + ib_v[sl]
            return carry

        lax.fori_loop(0, groups, group, 0)
        for d in range(dim):
            pltpu.sync_copy(ie_v.at[d], ie_out_hbm.at[d, pl.ds(base, bpw)])
        pltpu.sync_copy(ub_v, bias_out_hbm.at[pl.ds(base, bpw)])

    return k1


@functools.lru_cache(maxsize=None)
def _build_k2(batch, dim, uvocab):
    info = plsc.get_sparse_core_info()
    nc, ns = info.num_cores, info.num_subcores
    nw = nc * ns
    bpw = batch // nw
    groups = bpw // _LANES
    unb = uvocab // 128
    utrunc = unb * 128
    utail = uvocab - utrunc

    @functools.partial(
        pl.kernel,
        mesh=_mesh(),
        compiler_params=_PARAMS,
        out_type=jax.ShapeDtypeStruct((batch,), jnp.float32),
        scratch_types=[
            pltpu.VMEM((bpw,), jnp.int32),
            pltpu.VMEM((bpw,), jnp.int32),
            pltpu.VMEM((dim, bpw), jnp.float32),
            pltpu.VMEM((dim, bpw), jnp.float32),
            pltpu.VMEM((max(utail, 8), dim), jnp.float32),
            pltpu.VMEM((bpw,), jnp.float32),
            pltpu.VMEM((bpw,), jnp.float32),
            pltpu.SemaphoreType.DMA,
        ],
    )
    def k2(user_hbm, uflat_hbm, uside_hbm, ie_hbm, bias_hbm, out_hbm,
           uidx_v, uoff_v, ue_v, ie_v, uside_v, bs_v, out_v, sem):
        wid = lax.axis_index("s") * nc + lax.axis_index("c")
        base = wid * bpw
        pltpu.sync_copy(user_hbm.at[pl.ds(base, bpw)], uidx_v)

        def offsets(g, _):
            sl = pl.ds(g * _LANES, _LANES)
            ru = jnp.minimum(uidx_v[sl], utrunc - 1)
            uoff_v[sl] = (ru >> 7) * 1024 + (ru & 127)
            return _

        lax.fori_loop(0, groups, offsets, 0)

        copies = [
            pltpu.async_copy(uside_hbm, uside_v.at[pl.ds(0, utail)], sem),
            pltpu.async_copy(bias_hbm.at[pl.ds(base, bpw)], bs_v, sem),
        ]
        for d in range(dim):
            cu = (d // 8) * unb * 1024 + (d % 8) * 128
            span_u = unb * 1024 - (d % 8) * 128
            copies.append(pltpu.async_copy(
                uflat_hbm.at[pl.ds(cu, span_u)].at[uoff_v], ue_v.at[d], sem))
            copies.append(pltpu.async_copy(
                ie_hbm.at[d, pl.ds(base, bpw)], ie_v.at[d], sem))
        for cp in copies:
            cp.wait()

        def group(g, carry):
            sl = pl.ds(g * _LANES, _LANES)
            ru = uidx_v[sl]
            umask = ru >= utrunc
            su = jnp.where(umask, ru - utrunc, 0)
            acc = bs_v[sl]
            for d in range(dim):
                dcol = jnp.full((_LANES,), d, jnp.int32)
                uval = jnp.where(umask, plsc.load_gather(uside_v, [su, dcol]),
                                 ue_v[d, sl])
                acc = acc + uval * ie_v[d, sl]
            out_v[sl] = acc
            return carry

        lax.fori_loop(0, groups, group, 0)
        pltpu.sync_copy(out_v, out_hbm.at[pl.ds(base, bpw)])

    return k2


def kernel(user, item, user_emb_w, item_emb_w, user_bias_w, item_bias_w):
    batch = user.shape[0]
    uvocab, dim = user_emb_w.shape
    ivocab = item_emb_w.shape[0]
    user = user.astype(jnp.int32)
    item = item.astype(jnp.int32)
    ie_staged, bias_sum = _build_k1(batch, dim, ivocab)(
        user, item, _flat_view(item_emb_w),
        item_emb_w[(ivocab // 128) * 128:, :],
        user_bias_w.T, item_bias_w.T)
    return _build_k2(batch, dim, uvocab)(
        user, _flat_view(user_emb_w),
        user_emb_w[(uvocab // 128) * 128:, :],
        ie_staged, bias_sum)
